# baseline (speedup 1.0000x reference)
import jax
import jax.numpy as jnp
from jax import lax
from jax.experimental import pallas as pl
from jax.experimental.pallas import tpu as pltpu
from jax.experimental.pallas import tpu_sc as plsc

B = 4
SEQ = 2048
EMB = 768
EPS = 1e-6

NC = 2
NS = 16
LANES = 16
NW = NC * NS
N_ROWS = B * SEQ
ROWS_PER_W = N_ROWS // NW
C = 32
N_CHUNKS = ROWS_PER_W // C
HCHUNKS = EMB // LANES
UNROLL = 8
NBUF = 2

_DNUMS = lax.GatherDimensionNumbers(
    offset_dims=(), collapsed_slice_dims=(0,), start_index_map=(0,))


def _shuffle(x, perm):
    return lax.gather(x, perm[:, None], _DNUMS, slice_sizes=(1,),
                      mode=lax.GatherScatterMode.PROMISE_IN_BOUNDS)


def _lane_sum(x):
    for sh in (8, 4, 2, 1):
        x = x + _shuffle(x, lax.iota(jnp.int32, 16) ^ sh)
    return x


def _rsqrt_newton(v):
    i = plsc.bitcast(v, jnp.int32)
    i = jnp.int32(0x5F3759DF) - (i >> 1)
    y = plsc.bitcast(i, jnp.float32)
    half_v = 0.5 * v
    for _ in range(2):
        y = y * (1.5 - half_v * y * y)
    return y


def _compute_chunk(k, tok_buf, px_buf, seg_res, s_ids, scale_buf,
                   bias_buf):
    def row_body(i, _):
        zeros = jnp.zeros((LANES,), jnp.float32)
        s_vec = s_ids[pl.ds(k * C + (i & ~15), LANES)]
        s_splat = _shuffle(s_vec, jnp.full((LANES,), i & 15, jnp.int32))
        m1 = s_splat == 1
        m2 = s_splat == 2

        def acc_body(c, carry):
            acc, acc2 = carry
            col = c * LANES
            t = tok_buf[i, pl.ds(col, LANES)]
            p = px_buf[i, pl.ds(col, LANES)]
            r1 = seg_res[1, pl.ds(col, LANES)]
            r2 = seg_res[2, pl.ds(col, LANES)]
            g = jnp.where(m1, r1, zeros)
            g = jnp.where(m2, r2, g)
            x = t + p + g
            px_buf[i, pl.ds(col, LANES)] = x
            return acc + x, acc2 + x * x

        acc, acc2 = plsc.parallel_loop(
            0, HCHUNKS, unroll=UNROLL, carry=(zeros, zeros))(acc_body)
        tot_v = _lane_sum(acc)
        tot2_v = _lane_sum(acc2)
        mean_v = tot_v * (1.0 / EMB)
        var_v = (tot2_v - tot_v * mean_v) * (1.0 / (EMB - 1))
        std_v = var_v * _rsqrt_newton(var_v)
        std_v = jnp.where(var_v > 0.0, std_v, 0.0)
        r_v = 1.0 / (std_v + EPS)

        def norm_body(c):
            col = c * LANES
            x = px_buf[i, pl.ds(col, LANES)]
            sc = scale_buf[pl.ds(col, LANES)]
            bs = bias_buf[pl.ds(col, LANES)]
            px_buf[i, pl.ds(col, LANES)] = (x - mean_v) * r_v * sc + bs

        plsc.parallel_loop(0, HCHUNKS, unroll=UNROLL)(norm_body)
        return 0

    lax.fori_loop(0, C, row_body, 0)


def _sc_body(sentence_hbm, seg_label_hbm, tok_hbm, pos_hbm, seg_hbm,
             scale_hbm, bias_hbm, out_hbm,
             idx_all, s_ids, tok_buf, px_buf, seg_res,
             scale_buf, bias_buf, gsem, osem):
    wid = lax.axis_index("s") * NC + lax.axis_index("c")
    row0 = wid * ROWS_PER_W
    b = row0 // SEQ
    l0 = row0 % SEQ

    pltpu.sync_copy(scale_hbm, scale_buf)
    pltpu.sync_copy(bias_hbm, bias_buf)
    pltpu.sync_copy(seg_hbm, seg_res)
    pltpu.sync_copy(sentence_hbm.at[b, pl.ds(l0, ROWS_PER_W)], idx_all)
    pltpu.sync_copy(seg_label_hbm.at[b, pl.ds(l0, ROWS_PER_W)], s_ids)

    def issue_in(k):
        p = k % NBUF
        lc = l0 + k * C
        pltpu.async_copy(tok_hbm.at[idx_all.at[pl.ds(k * C, C)]],
                         tok_buf.at[p], gsem)
        pltpu.async_copy(pos_hbm.at[pl.ds(lc, C)], px_buf.at[p], gsem)

    def wait_in(k):
        p = k % NBUF
        pltpu.make_async_copy(tok_hbm.at[idx_all.at[pl.ds(k * C, C)]],
                              tok_buf.at[p], gsem).wait()
        pltpu.make_async_copy(pos_hbm.at[pl.ds(l0, C)], px_buf.at[p],
                              gsem).wait()

    def issue_out(k):
        p = k % NBUF
        pltpu.async_copy(px_buf.at[p], out_hbm.at[b, pl.ds(l0 + k * C, C)],
                         osem)

    def wait_out(k):
        p = k % NBUF
        pltpu.make_async_copy(px_buf.at[p],
                              out_hbm.at[b, pl.ds(l0 + k * C, C)],
                              osem).wait()

    issue_in(0)
    for k in range(N_CHUNKS):
        if k + 1 < N_CHUNKS:
            if k >= 1:
                wait_out(k - 1)
            issue_in(k + 1)
        wait_in(k)
        p = k % NBUF
        _compute_chunk(k, tok_buf.at[p], px_buf.at[p],
                       seg_res, s_ids, scale_buf, bias_buf)
        issue_out(k)
    for k in (N_CHUNKS - 2, N_CHUNKS - 1):
        wait_out(k)


@jax.jit
def _run(sentence, segment_label, tok_table, pos_table, seg_table,
         scale, bias):
    mesh = plsc.VectorSubcoreMesh(core_axis_name="c", subcore_axis_name="s")
    f = pl.kernel(
        _sc_body,
        out_type=jax.ShapeDtypeStruct((B, SEQ, EMB), jnp.float32),
        mesh=mesh,
        compiler_params=pltpu.CompilerParams(needs_layout_passes=False),
        scratch_types=[
            pltpu.VMEM((ROWS_PER_W,), jnp.int32),
            pltpu.VMEM((ROWS_PER_W,), jnp.int32),
            pltpu.VMEM((NBUF, C, EMB), jnp.float32),
            pltpu.VMEM((NBUF, C, EMB), jnp.float32),
            pltpu.VMEM((3, EMB), jnp.float32),
            pltpu.VMEM((EMB,), jnp.float32),
            pltpu.VMEM((EMB,), jnp.float32),
            pltpu.SemaphoreType.DMA,
            pltpu.SemaphoreType.DMA,
        ],
    )
    return f(sentence, segment_label, tok_table, pos_table, seg_table,
             scale, bias)


def kernel(sentence, segment_label, tok_table, pos_table, seg_table,
           scale, bias):
    return _run(sentence.astype(jnp.int32), segment_label.astype(jnp.int32),
                tok_table, pos_table, seg_table, scale, bias)


# speedup vs baseline: 1.2554x; 1.2554x over previous
import jax
import jax.numpy as jnp
from jax import lax
from jax.experimental import pallas as pl
from jax.experimental.pallas import tpu as pltpu
from jax.experimental.pallas import tpu_sc as plsc

B = 4
SEQ = 2048
EMB = 768
EPS = 1e-6

NC = 2
NS = 16
LANES = 16
NW = NC * NS
N_ROWS = B * SEQ
ROWS_PER_W = N_ROWS // NW
C = 16
N_CHUNKS = ROWS_PER_W // C
HCHUNKS = EMB // LANES
UNROLL = 8
NBUF = 2

_DNUMS = lax.GatherDimensionNumbers(
    offset_dims=(), collapsed_slice_dims=(0,), start_index_map=(0,))


def _shuffle(x, perm):
    return lax.gather(x, perm[:, None], _DNUMS, slice_sizes=(1,),
                      mode=lax.GatherScatterMode.PROMISE_IN_BOUNDS)


def _lane_sum(x):
    for sh in (8, 4, 2, 1):
        x = x + _shuffle(x, lax.iota(jnp.int32, 16) ^ sh)
    return x


def _rsqrt_newton(v):
    i = plsc.bitcast(v, jnp.int32)
    i = jnp.int32(0x5F3759DF) - (i >> 1)
    y = plsc.bitcast(i, jnp.float32)
    half_v = 0.5 * v
    for _ in range(2):
        y = y * (1.5 - half_v * y * y)
    return y


def _stats(acc, acc2):
    tot_v = _lane_sum(acc)
    tot2_v = _lane_sum(acc2)
    mean_v = tot_v * (1.0 / EMB)
    var_v = (tot2_v - tot_v * mean_v) * (1.0 / (EMB - 1))
    std_v = var_v * _rsqrt_newton(var_v)
    std_v = jnp.where(var_v > 0.0, std_v, 0.0)
    r_v = 1.0 / (std_v + EPS)
    return mean_v, r_v


def _compute_chunk(k, tok_buf, pos_buf, xout, seg_res, s_ids, scale_buf,
                   bias_buf):
    s_all = s_ids[pl.ds(k * C, LANES)]

    def pair_body(j, _):
        i0 = 2 * j
        i1 = i0 + 1
        zeros = jnp.zeros((LANES,), jnp.float32)
        s0 = _shuffle(s_all, jnp.full((LANES,), i0, jnp.int32))
        s1 = _shuffle(s_all, jnp.full((LANES,), i1, jnp.int32))
        m1a, m2a = s0 == 1, s0 == 2
        m1b, m2b = s1 == 1, s1 == 2

        def acc_body(c, carry):
            acc0, acc20, acc1, acc21 = carry
            col = c * LANES
            r1 = seg_res[1, pl.ds(col, LANES)]
            r2 = seg_res[2, pl.ds(col, LANES)]
            t0 = tok_buf[i0, pl.ds(col, LANES)]
            p0 = pos_buf[i0, pl.ds(col, LANES)]
            t1 = tok_buf[i1, pl.ds(col, LANES)]
            p1 = pos_buf[i1, pl.ds(col, LANES)]
            g0 = jnp.where(m1a, r1, zeros)
            g0 = jnp.where(m2a, r2, g0)
            g1 = jnp.where(m1b, r1, zeros)
            g1 = jnp.where(m2b, r2, g1)
            x0 = t0 + p0 + g0
            x1 = t1 + p1 + g1
            xout[i0, pl.ds(col, LANES)] = x0
            xout[i1, pl.ds(col, LANES)] = x1
            return acc0 + x0, acc20 + x0 * x0, acc1 + x1, acc21 + x1 * x1

        acc0, acc20, acc1, acc21 = plsc.parallel_loop(
            0, HCHUNKS, unroll=UNROLL,
            carry=(zeros, zeros, zeros, zeros))(acc_body)
        mean0, r0 = _stats(acc0, acc20)
        mean1, r1v = _stats(acc1, acc21)

        def norm_body(c):
            col = c * LANES
            sc = scale_buf[pl.ds(col, LANES)]
            bs = bias_buf[pl.ds(col, LANES)]
            x0 = xout[i0, pl.ds(col, LANES)]
            x1 = xout[i1, pl.ds(col, LANES)]
            xout[i0, pl.ds(col, LANES)] = (x0 - mean0) * r0 * sc + bs
            xout[i1, pl.ds(col, LANES)] = (x1 - mean1) * r1v * sc + bs

        plsc.parallel_loop(0, HCHUNKS, unroll=UNROLL)(norm_body)
        return 0

    lax.fori_loop(0, C // 2, pair_body, 0)


def _sc_body(sentence_hbm, seg_label_hbm, tok_hbm, pos_hbm, seg_hbm,
             scale_hbm, bias_hbm, out_hbm,
             idx_all, s_ids, tok_buf, pos_buf, xout, seg_res,
             scale_buf, bias_buf, gsem, osem):
    wid = lax.axis_index("s") * NC + lax.axis_index("c")
    row0 = wid * ROWS_PER_W
    b = row0 // SEQ
    l0 = row0 % SEQ

    pltpu.sync_copy(scale_hbm, scale_buf)
    pltpu.sync_copy(bias_hbm, bias_buf)
    pltpu.sync_copy(seg_hbm, seg_res)
    pltpu.sync_copy(sentence_hbm.at[b, pl.ds(l0, ROWS_PER_W)], idx_all)
    pltpu.sync_copy(seg_label_hbm.at[b, pl.ds(l0, ROWS_PER_W)], s_ids)

    def issue_in(k):
        p = k % NBUF
        lc = l0 + k * C
        pltpu.async_copy(tok_hbm.at[idx_all.at[pl.ds(k * C, C)]],
                         tok_buf.at[p], gsem)
        pltpu.async_copy(pos_hbm.at[pl.ds(lc, C)], pos_buf.at[p], gsem)

    def wait_in(k):
        p = k % NBUF
        pltpu.make_async_copy(tok_hbm.at[idx_all.at[pl.ds(k * C, C)]],
                              tok_buf.at[p], gsem).wait()
        pltpu.make_async_copy(pos_hbm.at[pl.ds(l0, C)], pos_buf.at[p],
                              gsem).wait()

    def issue_out(k):
        p = k % NBUF
        pltpu.async_copy(xout.at[p], out_hbm.at[b, pl.ds(l0 + k * C, C)],
                         osem)

    def wait_out(k):
        p = k % NBUF
        pltpu.make_async_copy(xout.at[p],
                              out_hbm.at[b, pl.ds(l0 + k * C, C)],
                              osem).wait()

    issue_in(0)
    for k in range(N_CHUNKS):
        if k + 1 < N_CHUNKS:
            issue_in(k + 1)
        wait_in(k)
        if k >= 2:
            wait_out(k - 2)
        p = k % NBUF
        _compute_chunk(k, tok_buf.at[p], pos_buf.at[p], xout.at[p],
                       seg_res, s_ids, scale_buf, bias_buf)
        issue_out(k)
    for k in (N_CHUNKS - 2, N_CHUNKS - 1):
        wait_out(k)


@jax.jit
def _run(sentence, segment_label, tok_table, pos_table, seg_table,
         scale, bias):
    mesh = plsc.VectorSubcoreMesh(core_axis_name="c", subcore_axis_name="s")
    f = pl.kernel(
        _sc_body,
        out_type=jax.ShapeDtypeStruct((B, SEQ, EMB), jnp.float32),
        mesh=mesh,
        compiler_params=pltpu.CompilerParams(needs_layout_passes=False),
        scratch_types=[
            pltpu.VMEM((ROWS_PER_W,), jnp.int32),
            pltpu.VMEM((ROWS_PER_W,), jnp.int32),
            pltpu.VMEM((NBUF, C, EMB), jnp.float32),
            pltpu.VMEM((NBUF, C, EMB), jnp.float32),
            pltpu.VMEM((NBUF, C, EMB), jnp.float32),
            pltpu.VMEM((3, EMB), jnp.float32),
            pltpu.VMEM((EMB,), jnp.float32),
            pltpu.VMEM((EMB,), jnp.float32),
            pltpu.SemaphoreType.DMA,
            pltpu.SemaphoreType.DMA,
        ],
    )
    return f(sentence, segment_label, tok_table, pos_table, seg_table,
             scale, bias)


def kernel(sentence, segment_label, tok_table, pos_table, seg_table,
           scale, bias):
    return _run(sentence.astype(jnp.int32), segment_label.astype(jnp.int32),
                tok_table, pos_table, seg_table, scale, bias)


# speedup vs baseline: 1.4759x; 1.1757x over previous
import jax
import jax.numpy as jnp
from jax import lax
from jax.experimental import pallas as pl
from jax.experimental.pallas import tpu as pltpu
from jax.experimental.pallas import tpu_sc as plsc

B = 4
SEQ = 2048
EMB = 768
EPS = 1e-6

NC = 2
NS = 16
LANES = 16
NW = NC * NS
N_ROWS = B * SEQ
ROWS_PER_W = N_ROWS // NW
C = 16
N_CHUNKS = ROWS_PER_W // C
HCHUNKS = EMB // LANES
UNROLL = 8
NBUF = 2

_DNUMS = lax.GatherDimensionNumbers(
    offset_dims=(), collapsed_slice_dims=(0,), start_index_map=(0,))


def _shuffle(x, perm):
    return lax.gather(x, perm[:, None], _DNUMS, slice_sizes=(1,),
                      mode=lax.GatherScatterMode.PROMISE_IN_BOUNDS)


def _lane_sum(x):
    for sh in (8, 4, 2, 1):
        x = x + _shuffle(x, lax.iota(jnp.int32, 16) ^ sh)
    return x


def _rsqrt_newton(v):
    i = plsc.bitcast(v, jnp.int32)
    i = jnp.int32(0x5F3759DF) - (i >> 1)
    y = plsc.bitcast(i, jnp.float32)
    half_v = 0.5 * v
    for _ in range(2):
        y = y * (1.5 - half_v * y * y)
    return y


def _stats(acc, acc2):
    tot_v = _lane_sum(acc)
    tot2_v = _lane_sum(acc2)
    mean_v = tot_v * (1.0 / EMB)
    var_v = (tot2_v - tot_v * mean_v) * (1.0 / (EMB - 1))
    std_v = var_v * _rsqrt_newton(var_v)
    std_v = jnp.where(var_v > 0.0, std_v, 0.0)
    r_v = 1.0 / (std_v + EPS)
    return mean_v, r_v


def _compute_chunk(k, tok_buf, pos_buf, xout, seg_res, s_ids, scale_buf,
                   bias_buf):
    s_all = s_ids[pl.ds(k * C, LANES)]

    NR = 4

    def quad_body(j, _):
        rows = [NR * j + d for d in range(NR)]
        zeros = jnp.zeros((LANES,), jnp.float32)
        masks = []
        for i in rows:
            s = _shuffle(s_all, jnp.full((LANES,), i, jnp.int32))
            masks.append((s == 1, s == 2))

        def acc_body(c, carry):
            col = c * LANES
            r1 = seg_res[1, pl.ds(col, LANES)]
            r2 = seg_res[2, pl.ds(col, LANES)]
            out = []
            for d, i in enumerate(rows):
                t = tok_buf[i, pl.ds(col, LANES)]
                p = pos_buf[i, pl.ds(col, LANES)]
                m1, m2 = masks[d]
                g = jnp.where(m1, r1, zeros)
                g = jnp.where(m2, r2, g)
                x = t + p + g
                xout[i, pl.ds(col, LANES)] = x
                out.append(carry[2 * d] + x)
                out.append(carry[2 * d + 1] + x * x)
            return tuple(out)

        accs = plsc.parallel_loop(
            0, HCHUNKS, unroll=UNROLL, carry=(zeros,) * (2 * NR))(acc_body)
        stats = [_stats(accs[2 * d], accs[2 * d + 1]) for d in range(NR)]

        def norm_body(c):
            col = c * LANES
            sc = scale_buf[pl.ds(col, LANES)]
            bs = bias_buf[pl.ds(col, LANES)]
            for d, i in enumerate(rows):
                mean_v, r_v = stats[d]
                x = xout[i, pl.ds(col, LANES)]
                xout[i, pl.ds(col, LANES)] = (x - mean_v) * r_v * sc + bs

        plsc.parallel_loop(0, HCHUNKS, unroll=UNROLL)(norm_body)
        return 0

    lax.fori_loop(0, C // NR, quad_body, 0)


def _sc_body(sentence_hbm, seg_label_hbm, tok_hbm, pos_hbm, seg_hbm,
             scale_hbm, bias_hbm, out_hbm,
             idx_all, s_ids, tok_buf, pos_buf, xout, seg_res,
             scale_buf, bias_buf, gsem, osem):
    wid = lax.axis_index("s") * NC + lax.axis_index("c")
    row0 = wid * ROWS_PER_W
    b = row0 // SEQ
    l0 = row0 % SEQ

    pltpu.sync_copy(scale_hbm, scale_buf)
    pltpu.sync_copy(bias_hbm, bias_buf)
    pltpu.sync_copy(seg_hbm, seg_res)
    pltpu.sync_copy(sentence_hbm.at[b, pl.ds(l0, ROWS_PER_W)], idx_all)
    pltpu.sync_copy(seg_label_hbm.at[b, pl.ds(l0, ROWS_PER_W)], s_ids)

    def issue_in(k):
        p = k % NBUF
        lc = l0 + k * C
        pltpu.async_copy(tok_hbm.at[idx_all.at[pl.ds(k * C, C)]],
                         tok_buf.at[p], gsem)
        pltpu.async_copy(pos_hbm.at[pl.ds(lc, C)], pos_buf.at[p], gsem)

    def wait_in(k):
        p = k % NBUF
        pltpu.make_async_copy(tok_hbm.at[idx_all.at[pl.ds(k * C, C)]],
                              tok_buf.at[p], gsem).wait()
        pltpu.make_async_copy(pos_hbm.at[pl.ds(l0, C)], pos_buf.at[p],
                              gsem).wait()

    def issue_out(k):
        p = k % NBUF
        pltpu.async_copy(xout.at[p], out_hbm.at[b, pl.ds(l0 + k * C, C)],
                         osem)

    def wait_out(k):
        p = k % NBUF
        pltpu.make_async_copy(xout.at[p],
                              out_hbm.at[b, pl.ds(l0 + k * C, C)],
                              osem).wait()

    issue_in(0)

    def chunk_body(k, _):
        p = lax.rem(k, NBUF)

        @pl.when(k + 1 < N_CHUNKS)
        def _():
            issue_in(k + 1)

        wait_in(k)

        @pl.when(k >= 2)
        def _():
            wait_out(k - 2)

        _compute_chunk(k, tok_buf.at[p], pos_buf.at[p], xout.at[p],
                       seg_res, s_ids, scale_buf, bias_buf)
        issue_out(k)
        return 0

    lax.fori_loop(0, N_CHUNKS, chunk_body, 0)
    for k in (N_CHUNKS - 2, N_CHUNKS - 1):
        wait_out(k)


@jax.jit
def _run(sentence, segment_label, tok_table, pos_table, seg_table,
         scale, bias):
    mesh = plsc.VectorSubcoreMesh(core_axis_name="c", subcore_axis_name="s")
    f = pl.kernel(
        _sc_body,
        out_type=jax.ShapeDtypeStruct((B, SEQ, EMB), jnp.float32),
        mesh=mesh,
        compiler_params=pltpu.CompilerParams(needs_layout_passes=False),
        scratch_types=[
            pltpu.VMEM((ROWS_PER_W,), jnp.int32),
            pltpu.VMEM((ROWS_PER_W,), jnp.int32),
            pltpu.VMEM((NBUF, C, EMB), jnp.float32),
            pltpu.VMEM((NBUF, C, EMB), jnp.float32),
            pltpu.VMEM((NBUF, C, EMB), jnp.float32),
            pltpu.VMEM((3, EMB), jnp.float32),
            pltpu.VMEM((EMB,), jnp.float32),
            pltpu.VMEM((EMB,), jnp.float32),
            pltpu.SemaphoreType.DMA,
            pltpu.SemaphoreType.DMA,
        ],
    )
    return f(sentence, segment_label, tok_table, pos_table, seg_table,
             scale, bias)


def kernel(sentence, segment_label, tok_table, pos_table, seg_table,
           scale, bias):
    return _run(sentence.astype(jnp.int32), segment_label.astype(jnp.int32),
                tok_table, pos_table, seg_table, scale, bias)
